# Initial kernel scaffold; baseline (speedup 1.0000x reference)
import jax
import jax.numpy as jnp
from jax.experimental import pallas as pl


def kernel(x, edge_index, W1, b1, gamma1, beta1, rm1, rv1, W2, b2):
    raise NotImplementedError("write your pallas kernel here")


# speedup vs baseline: 9.2612x; 9.2612x over previous
import functools

import jax
import jax.numpy as jnp
from jax import lax
from jax.experimental import pallas as pl
from jax.experimental.pallas import tpu as pltpu
from jax.experimental.pallas import tpu_sc as plsc

_N = 10000
_D = 128
_H = 256
_C = 40
_E = 320000
_NP = 10240
_EP = 323584
_K = 128
_BN_EPS = 1e-5
_RPS = _NP // 16


def _sc_mesh():
    return plsc.VectorSubcoreMesh(
        core_axis_name="c", subcore_axis_name="s", num_cores=2, num_subcores=16
    )


def _deg_body(dst_hbm, zeros_hbm, degs_hbm, acc, vidx, vones):
    c = lax.axis_index("c")
    s = lax.axis_index("s")
    pltpu.sync_copy(zeros_hbm.at[pl.ds(s * _RPS, _RPS)], acc.at[pl.ds(s * _RPS, _RPS)])
    for i in range(_K // 16):
        vones[pl.ds(i * 16, 16)] = jnp.ones((16,), jnp.float32)
    plsc.subcore_barrier()
    w = c * 16 + s
    per_w = _EP // 32
    base = w * per_w

    def chunk(g, carry):
        off = base + g * _K
        pltpu.sync_copy(dst_hbm.at[pl.ds(off, _K)], vidx)
        pltpu.sync_copy(vones, acc.at[vidx], add=True)
        return carry

    lax.fori_loop(0, per_w // _K, chunk, 0)
    plsc.subcore_barrier()
    pltpu.sync_copy(acc.at[pl.ds(s * _RPS, _RPS)], degs_hbm.at[c, pl.ds(s * _RPS, _RPS)])


def _run_deg(dstp):
    k = pl.kernel(
        _deg_body,
        out_type=jax.ShapeDtypeStruct((2, _NP), jnp.float32),
        mesh=_sc_mesh(),
        scratch_types=[
            pltpu.VMEM_SHARED((_NP,), jnp.float32),
            pltpu.VMEM((_K,), jnp.int32),
            pltpu.VMEM((_K,), jnp.float32),
        ],
    )
    return k(dstp, jnp.zeros((_NP,), jnp.float32))


def _agg_body(F, hb_hbm, src2_hbm, dst_hbm, out_hbm, acc, sidx, didx, rowbuf, gsem):
    c = lax.axis_index("c")
    s = lax.axis_index("s")
    pltpu.sync_copy(
        hb_hbm.at[pl.ds(c * _NP + s * _RPS, _RPS)], acc.at[pl.ds(s * _RPS, _RPS)]
    )
    plsc.subcore_barrier()
    per_s = _EP // 16
    base = s * per_s

    def chunk(g, carry):
        off = base + g * _K
        pltpu.sync_copy(src2_hbm.at[c, pl.ds(off, _K)], sidx)
        pltpu.async_copy(hb_hbm.at[sidx], rowbuf, gsem).wait()
        pltpu.sync_copy(dst_hbm.at[pl.ds(off, _K)], didx)
        pltpu.sync_copy(rowbuf, acc.at[didx], add=True)
        return carry

    lax.fori_loop(0, per_s // _K, chunk, 0)
    plsc.subcore_barrier()
    pltpu.sync_copy(
        acc.at[pl.ds(s * _RPS, _RPS)], out_hbm.at[c, pl.ds(s * _RPS, _RPS)]
    )


def _run_agg(hb_flat, src2, dstp, F):
    k = pl.kernel(
        functools.partial(_agg_body, F),
        out_type=jax.ShapeDtypeStruct((2, _NP, F), jnp.float32),
        mesh=_sc_mesh(),
        scratch_types=[
            pltpu.VMEM_SHARED((_NP, F), jnp.float32),
            pltpu.VMEM((_K,), jnp.int32),
            pltpu.VMEM((_K,), jnp.int32),
            pltpu.VMEM((_K, F), jnp.float32),
            pltpu.SemaphoreType.DMA,
        ],
    )
    return k(hb_flat, src2, dstp)


def _agg2_body(hb_hbm, src_hbm, dst_hbm, out_hbm, acc, sidx, didx, rowbuf, gsem):
    c = lax.axis_index("c")
    s = lax.axis_index("s")
    pltpu.sync_copy(
        hb_hbm.at[pl.ds(c * _NP + s * _RPS, _RPS)], acc.at[pl.ds(s * _RPS, _RPS)]
    )
    plsc.subcore_barrier()
    w = c * 16 + s
    per_w = _EP // 32
    base = w * per_w

    def chunk(g, carry):
        off = base + g * _K
        pltpu.sync_copy(src_hbm.at[pl.ds(off, _K)], sidx)
        pltpu.async_copy(hb_hbm.at[sidx], rowbuf, gsem).wait()
        pltpu.sync_copy(dst_hbm.at[pl.ds(off, _K)], didx)
        pltpu.sync_copy(rowbuf, acc.at[didx], add=True)
        return carry

    lax.fori_loop(0, per_w // _K, chunk, 0)
    plsc.subcore_barrier()
    pltpu.sync_copy(
        acc.at[pl.ds(s * _RPS, _RPS)], out_hbm.at[c, pl.ds(s * _RPS, _RPS)]
    )


def _run_agg2(hbcat, srcp, dstp):
    k = pl.kernel(
        _agg2_body,
        out_type=jax.ShapeDtypeStruct((2, _NP, 128), jnp.float32),
        mesh=_sc_mesh(),
        scratch_types=[
            pltpu.VMEM_SHARED((_NP, 128), jnp.float32),
            pltpu.VMEM((_K,), jnp.int32),
            pltpu.VMEM((_K,), jnp.int32),
            pltpu.VMEM((_K, 128), jnp.float32),
            pltpu.SemaphoreType.DMA,
        ],
    )
    return k(hbcat, srcp, dstp)


_BN1 = 512


def _tc1_body(x_ref, w_ref, degs_ref, out_ref):
    deg = 1.0 + degs_ref[0, :] + degs_ref[1, :]
    dinv = lax.rsqrt(deg)
    h = jnp.dot(x_ref[...], w_ref[...], preferred_element_type=jnp.float32)
    out_ref[0] = h * dinv[:, None]


def _run_tc1(xp, W1, degs):
    return pl.pallas_call(
        _tc1_body,
        grid=(_NP // _BN1, 2),
        in_specs=[
            pl.BlockSpec((_BN1, _D), lambda i, j: (i, 0)),
            pl.BlockSpec((_D, _H // 2), lambda i, j: (0, j)),
            pl.BlockSpec((2, _BN1), lambda i, j: (0, i)),
        ],
        out_specs=pl.BlockSpec((1, _BN1, _H // 2), lambda i, j: (j, i, 0)),
        out_shape=jax.ShapeDtypeStruct((2, _NP, _H // 2), jnp.float32),
    )(xp, W1, degs)


_BN2 = 512


def _tc2_body(t_ref, degs_ref, a_ref, b_ref, w2_ref, out_ref):
    deg = 1.0 + degs_ref[0, :] + degs_ref[1, :]
    dinv = lax.rsqrt(deg)
    t = t_ref[...]
    h = t * dinv[None, :, None] * a_ref[...] + b_ref[...]
    h = jnp.maximum(h, 0.0)
    y = jnp.dot(h[0], w2_ref[0], preferred_element_type=jnp.float32)
    y = y + jnp.dot(h[1], w2_ref[1], preferred_element_type=jnp.float32)
    out_ref[...] = y * dinv[:, None]


def _run_tc2(tb1, degs, a_r, b_r, w2p):
    return pl.pallas_call(
        _tc2_body,
        grid=(_NP // _BN2,),
        in_specs=[
            pl.BlockSpec((2, _BN2, _H // 2), lambda i: (0, i, 0)),
            pl.BlockSpec((2, _BN2), lambda i: (0, i)),
            pl.BlockSpec((2, 1, _H // 2), lambda i: (0, 0, 0)),
            pl.BlockSpec((2, 1, _H // 2), lambda i: (0, 0, 0)),
            pl.BlockSpec((2, _H // 2, 128), lambda i: (0, 0, 0)),
        ],
        out_specs=pl.BlockSpec((_BN2, 128), lambda i: (i, 0)),
        out_shape=jax.ShapeDtypeStruct((_NP, 128), jnp.float32),
    )(tb1, degs, a_r, b_r, w2p)


_BN3 = 1024


def _tc3_body(t_ref, degs_ref, b2_ref, out_ref):
    deg = 1.0 + degs_ref[0, :] + degs_ref[1, :]
    dinv = lax.rsqrt(deg)
    t = t_ref[...]
    o = (t[0] + t[1])[:, : _C]
    o = o * dinv[:, None] + b2_ref[...]
    m = jnp.max(o, axis=1, keepdims=True)
    l = o - m
    lse = jnp.log(jnp.sum(jnp.exp(l), axis=1, keepdims=True))
    out_ref[...] = l - lse


def _run_tc3(tb2, degs, b2_r):
    return pl.pallas_call(
        _tc3_body,
        grid=(_NP // _BN3,),
        in_specs=[
            pl.BlockSpec((2, _BN3, 128), lambda i: (0, i, 0)),
            pl.BlockSpec((2, _BN3), lambda i: (0, i)),
            pl.BlockSpec((1, _C), lambda i: (0, 0)),
        ],
        out_specs=pl.BlockSpec((_BN3, _C), lambda i: (i, 0)),
        out_shape=jax.ShapeDtypeStruct((_NP, _C), jnp.float32),
    )(tb2, degs, b2_r)


def kernel(x, edge_index, W1, b1, gamma1, beta1, rm1, rv1, W2, b2):
    src, dst = edge_index[0], edge_index[1]
    pad = _EP - _E
    srcp = jnp.concatenate([src, jnp.full((pad,), _N, jnp.int32)])
    dstp = jnp.concatenate([dst, jnp.full((pad,), _N, jnp.int32)])
    src2 = jnp.stack([srcp, srcp + _NP])
    xp = jnp.concatenate([x, jnp.zeros((_NP - _N, _D), jnp.float32)])

    degs = _run_deg(dstp)

    hb1 = _run_tc1(xp, W1, degs)
    tb1 = _run_agg(hb1.reshape(2 * _NP, _H // 2), src2, dstp, _H // 2)

    a = gamma1 * lax.rsqrt(rv1 + _BN_EPS)
    bv = (b1 - rm1) * a + beta1
    w2p = jnp.pad(W2, ((0, 0), (0, 128 - _C))).reshape(2, _H // 2, 128)
    hb2 = _run_tc2(tb1, degs, a.reshape(2, 1, _H // 2), bv.reshape(2, 1, _H // 2),
                   w2p)
    hbcat = jnp.concatenate([hb2, jnp.zeros((_NP, 128), jnp.float32)])
    tb2 = _run_agg2(hbcat, srcp, dstp)

    out = _run_tc3(tb2, degs, b2.reshape(1, _C))
    return out[:_N]


# speedup vs baseline: 9.9370x; 1.0730x over previous
import jax
import jax.numpy as jnp
from jax import lax
from jax.experimental import pallas as pl
from jax.experimental.pallas import tpu as pltpu
from jax.experimental.pallas import tpu_sc as plsc

_N = 10000
_D = 128
_H = 256
_C = 40
_E = 320000
_NP = 10240
_EP = 327680
_K = 128
_NCH1 = _EP // 16 // _K
_NCH2 = _EP // 32 // _K
_BN_EPS = 1e-5
_RPS = _NP // 16


def _sc_mesh():
    return plsc.VectorSubcoreMesh(
        core_axis_name="c", subcore_axis_name="s", num_cores=2, num_subcores=16
    )


_BCH1 = 32
_BCH2 = 40


def _edge_loop(nbatch, bch, src_sel, dst_sel, hb_hbm, acc, sidx_all, didx_all,
               row0, row1, sem0, sem1):
    rows = (row0, row1)
    sems = (sem0, sem1)

    def batch(bi, carry):
        pltpu.sync_copy(src_sel(bi), sidx_all)
        pltpu.sync_copy(dst_sel(bi), didx_all)
        for b in range(2):
            pltpu.async_copy(hb_hbm.at[sidx_all.at[b]], rows[b], sems[b])

        def body(i, c2):
            for b in range(2):
                g = 2 * i + b
                pltpu.make_async_copy(
                    hb_hbm.at[sidx_all.at[g]], rows[b], sems[b]
                ).wait()
                pltpu.sync_copy(rows[b], acc.at[didx_all.at[g]], add=True)

                @pl.when(g + 2 < bch)
                def _issue():
                    pltpu.async_copy(
                        hb_hbm.at[sidx_all.at[g + 2]], rows[b], sems[b]
                    )

            return c2

        lax.fori_loop(0, bch // 2, body, 0)
        return carry

    lax.fori_loop(0, nbatch, batch, 0)


def _deg_body(dst_hbm, zeros_hbm, degs_hbm, acc, didx_all, vones):
    c = lax.axis_index("c")
    s = lax.axis_index("s")
    pltpu.sync_copy(zeros_hbm.at[pl.ds(s * _RPS, _RPS)], acc.at[pl.ds(s * _RPS, _RPS)])
    pltpu.sync_copy(dst_hbm.at[c, s], didx_all)
    for i in range(_K // 16):
        vones[pl.ds(i * 16, 16)] = jnp.ones((16,), jnp.float32)
    plsc.subcore_barrier()

    def chunk(g, carry):
        pltpu.sync_copy(vones, acc.at[didx_all.at[g]], add=True)
        return carry

    lax.fori_loop(0, _NCH2, chunk, 0)
    plsc.subcore_barrier()
    pltpu.sync_copy(acc.at[pl.ds(s * _RPS, _RPS)], degs_hbm.at[c, pl.ds(s * _RPS, _RPS)])


def _run_deg(dst_w):
    k = pl.kernel(
        _deg_body,
        out_type=jax.ShapeDtypeStruct((2, _NP), jnp.float32),
        mesh=_sc_mesh(),
        scratch_types=[
            pltpu.VMEM_SHARED((_NP,), jnp.float32),
            pltpu.VMEM((_NCH2, _K), jnp.int32),
            pltpu.VMEM((_K,), jnp.float32),
        ],
    )
    return k(dst_w, jnp.zeros((_NP,), jnp.float32))


def _agg1_body(hb_hbm, src2_hbm, dst_hbm, out_hbm, acc, sidx_all, didx_all,
               row0, row1, sem0, sem1):
    c = lax.axis_index("c")
    s = lax.axis_index("s")
    pltpu.sync_copy(
        hb_hbm.at[pl.ds(c * _NP + s * _RPS, _RPS)], acc.at[pl.ds(s * _RPS, _RPS)]
    )
    plsc.subcore_barrier()
    _edge_loop(_NCH1 // _BCH1, _BCH1,
               lambda bi: src2_hbm.at[c, s, bi],
               lambda bi: dst_hbm.at[s, bi],
               hb_hbm, acc, sidx_all, didx_all, row0, row1, sem0, sem1)
    plsc.subcore_barrier()
    pltpu.sync_copy(
        acc.at[pl.ds(s * _RPS, _RPS)], out_hbm.at[c, pl.ds(s * _RPS, _RPS)]
    )


def _run_agg1(hb_flat, src2_r, dst_r):
    k = pl.kernel(
        _agg1_body,
        out_type=jax.ShapeDtypeStruct((2, _NP, 128), jnp.float32),
        mesh=_sc_mesh(),
        scratch_types=[
            pltpu.VMEM_SHARED((_NP, 128), jnp.float32),
            pltpu.VMEM((_BCH1, _K), jnp.int32),
            pltpu.VMEM((_BCH1, _K), jnp.int32),
            pltpu.VMEM((_K, 128), jnp.float32),
            pltpu.VMEM((_K, 128), jnp.float32),
            pltpu.SemaphoreType.DMA,
            pltpu.SemaphoreType.DMA,
        ],
    )
    return k(hb_flat, src2_r, dst_r)


def _agg2_body(hb_hbm, src_hbm, dst_hbm, out_hbm, acc, sidx_all, didx_all,
               row0, row1, sem0, sem1):
    c = lax.axis_index("c")
    s = lax.axis_index("s")
    pltpu.sync_copy(
        hb_hbm.at[pl.ds(c * _NP + s * _RPS, _RPS)], acc.at[pl.ds(s * _RPS, _RPS)]
    )
    plsc.subcore_barrier()
    _edge_loop(_NCH2 // _BCH2, _BCH2,
               lambda bi: src_hbm.at[c, s, bi],
               lambda bi: dst_hbm.at[c, s, bi],
               hb_hbm, acc, sidx_all, didx_all, row0, row1, sem0, sem1)
    plsc.subcore_barrier()
    pltpu.sync_copy(
        acc.at[pl.ds(s * _RPS, _RPS)], out_hbm.at[c, pl.ds(s * _RPS, _RPS)]
    )


def _run_agg2(hbcat, src_r, dst_r):
    k = pl.kernel(
        _agg2_body,
        out_type=jax.ShapeDtypeStruct((2, _NP, 128), jnp.float32),
        mesh=_sc_mesh(),
        scratch_types=[
            pltpu.VMEM_SHARED((_NP, 128), jnp.float32),
            pltpu.VMEM((_BCH2, _K), jnp.int32),
            pltpu.VMEM((_BCH2, _K), jnp.int32),
            pltpu.VMEM((_K, 128), jnp.float32),
            pltpu.VMEM((_K, 128), jnp.float32),
            pltpu.SemaphoreType.DMA,
            pltpu.SemaphoreType.DMA,
        ],
    )
    return k(hbcat, src_r, dst_r)


_BN1 = 512


def _tc1_body(x_ref, w_ref, degs_ref, out_ref):
    deg = 1.0 + degs_ref[0, :] + degs_ref[1, :]
    dinv = lax.rsqrt(deg)
    h = jnp.dot(x_ref[...], w_ref[...], preferred_element_type=jnp.float32)
    out_ref[0] = h * dinv[:, None]


def _run_tc1(xp, W1, degs):
    return pl.pallas_call(
        _tc1_body,
        grid=(_NP // _BN1, 2),
        in_specs=[
            pl.BlockSpec((_BN1, _D), lambda i, j: (i, 0)),
            pl.BlockSpec((_D, _H // 2), lambda i, j: (0, j)),
            pl.BlockSpec((2, _BN1), lambda i, j: (0, i)),
        ],
        out_specs=pl.BlockSpec((1, _BN1, _H // 2), lambda i, j: (j, i, 0)),
        out_shape=jax.ShapeDtypeStruct((2, _NP, _H // 2), jnp.float32),
    )(xp, W1, degs)


_BN2 = 512


def _tc2_body(t_ref, degs_ref, a_ref, b_ref, w2_ref, out_ref):
    deg = 1.0 + degs_ref[0, :] + degs_ref[1, :]
    dinv = lax.rsqrt(deg)
    t = t_ref[...]
    h = t * dinv[None, :, None] * a_ref[...] + b_ref[...]
    h = jnp.maximum(h, 0.0)
    y = jnp.dot(h[0], w2_ref[0], preferred_element_type=jnp.float32)
    y = y + jnp.dot(h[1], w2_ref[1], preferred_element_type=jnp.float32)
    out_ref[...] = y * dinv[:, None]


def _run_tc2(tb1, degs, a_r, b_r, w2p):
    return pl.pallas_call(
        _tc2_body,
        grid=(_NP // _BN2,),
        in_specs=[
            pl.BlockSpec((2, _BN2, _H // 2), lambda i: (0, i, 0)),
            pl.BlockSpec((2, _BN2), lambda i: (0, i)),
            pl.BlockSpec((2, 1, _H // 2), lambda i: (0, 0, 0)),
            pl.BlockSpec((2, 1, _H // 2), lambda i: (0, 0, 0)),
            pl.BlockSpec((2, _H // 2, 128), lambda i: (0, 0, 0)),
        ],
        out_specs=pl.BlockSpec((_BN2, 128), lambda i: (i, 0)),
        out_shape=jax.ShapeDtypeStruct((_NP, 128), jnp.float32),
    )(tb1, degs, a_r, b_r, w2p)


_BN3 = 1024


def _tc3_body(t_ref, degs_ref, b2_ref, out_ref):
    deg = 1.0 + degs_ref[0, :] + degs_ref[1, :]
    dinv = lax.rsqrt(deg)
    t = t_ref[...]
    o = (t[0] + t[1])[:, : _C]
    o = o * dinv[:, None] + b2_ref[...]
    m = jnp.max(o, axis=1, keepdims=True)
    l = o - m
    lse = jnp.log(jnp.sum(jnp.exp(l), axis=1, keepdims=True))
    out_ref[...] = l - lse


def _run_tc3(tb2, degs, b2_r):
    return pl.pallas_call(
        _tc3_body,
        grid=(_NP // _BN3,),
        in_specs=[
            pl.BlockSpec((2, _BN3, 128), lambda i: (0, i, 0)),
            pl.BlockSpec((2, _BN3), lambda i: (0, i)),
            pl.BlockSpec((1, _C), lambda i: (0, 0)),
        ],
        out_specs=pl.BlockSpec((_BN3, _C), lambda i: (i, 0)),
        out_shape=jax.ShapeDtypeStruct((_NP, _C), jnp.float32),
    )(tb2, degs, b2_r)


def kernel(x, edge_index, W1, b1, gamma1, beta1, rm1, rv1, W2, b2):
    src, dst = edge_index[0], edge_index[1]
    pad = _EP - _E
    srcp = jnp.concatenate([src, jnp.full((pad,), _N, jnp.int32)])
    dstp = jnp.concatenate([dst, jnp.full((pad,), _N, jnp.int32)])
    src2_r = jnp.stack([srcp, srcp + _NP]).reshape(2, 16, _NCH1 // _BCH1, _BCH1, _K)
    dst1_r = dstp.reshape(16, _NCH1 // _BCH1, _BCH1, _K)
    srcw_r = srcp.reshape(2, 16, _NCH2 // _BCH2, _BCH2, _K)
    dstw_r = dstp.reshape(2, 16, _NCH2 // _BCH2, _BCH2, _K)
    dstd_r = dstp.reshape(2, 16, _NCH2, _K)
    xp = jnp.concatenate([x, jnp.zeros((_NP - _N, _D), jnp.float32)])

    degs = _run_deg(dstd_r)

    hb1 = _run_tc1(xp, W1, degs)
    tb1 = _run_agg1(hb1.reshape(2 * _NP, _H // 2), src2_r, dst1_r)

    a = gamma1 * lax.rsqrt(rv1 + _BN_EPS)
    bv = (b1 - rm1) * a + beta1
    w2p = jnp.pad(W2, ((0, 0), (0, 128 - _C))).reshape(2, _H // 2, 128)
    hb2 = _run_tc2(tb1, degs, a.reshape(2, 1, _H // 2), bv.reshape(2, 1, _H // 2),
                   w2p)
    hbcat = jnp.concatenate([hb2, jnp.zeros((_NP, 128), jnp.float32)])
    tb2 = _run_agg2(hbcat, srcw_r, dstw_r)

    out = _run_tc3(tb2, degs, b2.reshape(1, _C))
    return out[:_N]


# speedup vs baseline: 33.2503x; 3.3461x over previous
import jax
import jax.numpy as jnp
from jax import lax
from jax.experimental import pallas as pl
from jax.experimental.pallas import tpu as pltpu
from jax.experimental.pallas import tpu_sc as plsc

_N = 10000
_D = 128
_H = 256
_C = 40
_E = 320000
_NP = 10240
_EP = 327680
_K = 128
_NCH = _EP // 32 // _K
_BCH = 40
_BN_EPS = 1e-5
_RPS = _NP // 16


def _sc_mesh():
    return plsc.VectorSubcoreMesh(
        core_axis_name="c", subcore_axis_name="s", num_cores=2, num_subcores=16
    )


def _deg_body(dst_hbm, zeros_hbm, degs_hbm, acc, didx_all, vones):
    c = lax.axis_index("c")
    s = lax.axis_index("s")
    pltpu.sync_copy(zeros_hbm.at[pl.ds(s * _RPS, _RPS)], acc.at[pl.ds(s * _RPS, _RPS)])
    pltpu.sync_copy(dst_hbm.at[c, s], didx_all)
    for i in range(_K // 16):
        vones[pl.ds(i * 16, 16)] = jnp.ones((16,), jnp.float32)
    plsc.subcore_barrier()

    def chunk(g, carry):
        pltpu.sync_copy(vones, acc.at[didx_all.at[g]], add=True)
        return carry

    lax.fori_loop(0, _NCH, chunk, 0)
    plsc.subcore_barrier()
    pltpu.sync_copy(acc.at[pl.ds(s * _RPS, _RPS)], degs_hbm.at[c, pl.ds(s * _RPS, _RPS)])


def _run_deg(dst_r):
    k = pl.kernel(
        _deg_body,
        out_type=jax.ShapeDtypeStruct((2, _NP), jnp.float32),
        mesh=_sc_mesh(),
        scratch_types=[
            pltpu.VMEM_SHARED((_NP,), jnp.float32),
            pltpu.VMEM((_NCH, _K), jnp.int32),
            pltpu.VMEM((_K,), jnp.float32),
        ],
    )
    return k(dst_r, jnp.zeros((_NP,), jnp.float32))


def _agg_body(hb_hbm, src_hbm, dst_hbm, out_hbm, acc, sidx_all, didx_all,
              row0, row1, sem0, sem1):
    c = lax.axis_index("c")
    s = lax.axis_index("s")
    pltpu.sync_copy(
        hb_hbm.at[pl.ds(c * _NP + s * _RPS, _RPS)], acc.at[pl.ds(s * _RPS, _RPS)]
    )
    plsc.subcore_barrier()
    rows = (row0, row1)
    sems = (sem0, sem1)

    def batch(bi, carry):
        pltpu.sync_copy(src_hbm.at[c, s, bi], sidx_all)
        pltpu.sync_copy(dst_hbm.at[c, s, bi], didx_all)
        for b in range(2):
            pltpu.async_copy(hb_hbm.at[sidx_all.at[b]], rows[b], sems[b])

        def body(i, c2):
            for b in range(2):
                g = 2 * i + b
                pltpu.make_async_copy(
                    hb_hbm.at[sidx_all.at[g]], rows[b], sems[b]
                ).wait()
                pltpu.sync_copy(rows[b], acc.at[didx_all.at[g]], add=True)

                @pl.when(g + 2 < _BCH)
                def _issue():
                    pltpu.async_copy(
                        hb_hbm.at[sidx_all.at[g + 2]], rows[b], sems[b]
                    )

            return c2

        lax.fori_loop(0, _BCH // 2, body, 0)
        return carry

    lax.fori_loop(0, _NCH // _BCH, batch, 0)
    plsc.subcore_barrier()
    pltpu.sync_copy(
        acc.at[pl.ds(s * _RPS, _RPS)], out_hbm.at[c, pl.ds(s * _RPS, _RPS)]
    )


def _run_agg(hbcat, src_r, dst_r):
    k = pl.kernel(
        _agg_body,
        out_type=jax.ShapeDtypeStruct((2, _NP, 128), jnp.float32),
        mesh=_sc_mesh(),
        scratch_types=[
            pltpu.VMEM_SHARED((_NP, 128), jnp.float32),
            pltpu.VMEM((_BCH, _K), jnp.int32),
            pltpu.VMEM((_BCH, _K), jnp.int32),
            pltpu.VMEM((_K, 128), jnp.float32),
            pltpu.VMEM((_K, 128), jnp.float32),
            pltpu.SemaphoreType.DMA,
            pltpu.SemaphoreType.DMA,
        ],
    )
    return k(hbcat, src_r, dst_r)


_BN = 512


def _tca_body(x_ref, degs_ref, out_ref):
    deg = 1.0 + degs_ref[0, :] + degs_ref[1, :]
    dinv = lax.rsqrt(deg)
    out_ref[...] = x_ref[...] * dinv[:, None]


def _run_tca(xp, degs):
    return pl.pallas_call(
        _tca_body,
        grid=(_NP // _BN,),
        in_specs=[
            pl.BlockSpec((_BN, _D), lambda i: (i, 0)),
            pl.BlockSpec((2, _BN), lambda i: (0, i)),
        ],
        out_specs=pl.BlockSpec((_BN, _D), lambda i: (i, 0)),
        out_shape=jax.ShapeDtypeStruct((_NP, _D), jnp.float32),
    )(xp, degs)


def _tcb_body(p_ref, degs_ref, a_ref, bv_ref, w1_ref, w2_ref, out_ref):
    i = pl.program_id(0)
    deg = 1.0 + degs_ref[0, :] + degs_ref[1, :]
    dinv = lax.rsqrt(deg)
    u = (p_ref[0] + p_ref[1]) * dinv[:, None]
    h = jnp.dot(u, w1_ref[...], preferred_element_type=jnp.float32)
    h = jnp.maximum(h * a_ref[...] + bv_ref[...], 0.0)
    y = jnp.dot(h, w2_ref[...], preferred_element_type=jnp.float32)
    y = y * dinv[:, None]
    rowid = i * _BN + lax.broadcasted_iota(jnp.int32, (_BN, 1), 0)
    out_ref[...] = jnp.where(rowid < _N, y, 0.0)


def _run_tcb(p, degs, a_r, bv_r, W1, w2p):
    return pl.pallas_call(
        _tcb_body,
        grid=(_NP // _BN,),
        in_specs=[
            pl.BlockSpec((2, _BN, _D), lambda i: (0, i, 0)),
            pl.BlockSpec((2, _BN), lambda i: (0, i)),
            pl.BlockSpec((1, _H), lambda i: (0, 0)),
            pl.BlockSpec((1, _H), lambda i: (0, 0)),
            pl.BlockSpec((_D, _H), lambda i: (0, 0)),
            pl.BlockSpec((_H, 128), lambda i: (0, 0)),
        ],
        out_specs=pl.BlockSpec((_BN, 128), lambda i: (i, 0)),
        out_shape=jax.ShapeDtypeStruct((_NP, 128), jnp.float32),
    )(p, degs, a_r, bv_r, W1, w2p)


_BN3 = 1024


def _tcc_body(t_ref, degs_ref, b2_ref, out_ref):
    deg = 1.0 + degs_ref[0, :] + degs_ref[1, :]
    dinv = lax.rsqrt(deg)
    t = t_ref[...]
    o = (t[0] + t[1])[:, : _C]
    o = o * dinv[:, None] + b2_ref[...]
    m = jnp.max(o, axis=1, keepdims=True)
    l = o - m
    lse = jnp.log(jnp.sum(jnp.exp(l), axis=1, keepdims=True))
    out_ref[...] = l - lse


def _run_tcc(q, degs, b2_r):
    return pl.pallas_call(
        _tcc_body,
        grid=(_NP // _BN3,),
        in_specs=[
            pl.BlockSpec((2, _BN3, 128), lambda i: (0, i, 0)),
            pl.BlockSpec((2, _BN3), lambda i: (0, i)),
            pl.BlockSpec((1, _C), lambda i: (0, 0)),
        ],
        out_specs=pl.BlockSpec((_BN3, _C), lambda i: (i, 0)),
        out_shape=jax.ShapeDtypeStruct((_NP, _C), jnp.float32),
    )(q, degs, b2_r)


def kernel(x, edge_index, W1, b1, gamma1, beta1, rm1, rv1, W2, b2):
    src, dst = edge_index[0], edge_index[1]
    pad = _EP - _E
    ip = jnp.arange(pad, dtype=jnp.int32)
    src_pad = _N + ip % (_NP - _N)
    srcp = jnp.concatenate([src, src_pad])
    dstp = jnp.concatenate([dst, (ip * 1009) % _NP])
    dstd = jnp.concatenate([dst, _N + ip % (_NP - _N)])
    src_r = srcp.reshape(2, 16, _NCH // _BCH, _BCH, _K)
    dst_r = dstp.reshape(2, 16, _NCH // _BCH, _BCH, _K)
    dstd_r = dstd.reshape(2, 16, _NCH, _K)
    xp = jnp.concatenate([x, jnp.zeros((_NP - _N, _D), jnp.float32)])
    zblock = jnp.zeros((_NP, 128), jnp.float32)

    degs = _run_deg(dstd_r)

    xhat = _run_tca(xp, degs)
    p = _run_agg(jnp.concatenate([xhat, zblock]), src_r, dst_r)

    a = gamma1 * lax.rsqrt(rv1 + _BN_EPS)
    bv = (b1 - rm1) * a + beta1
    w2p = jnp.pad(W2, ((0, 0), (0, 128 - _C)))
    yhat = _run_tcb(p, degs, a.reshape(1, _H), bv.reshape(1, _H), W1, w2p)
    q = _run_agg(jnp.concatenate([yhat, zblock]), src_r, dst_r)

    out = _run_tcc(q, degs, b2.reshape(1, _C))
    return out[:_N]
